# baseline (speedup 1.0000x reference)
import functools

import numpy as np
import jax
import jax.numpy as jnp
from jax import lax
from jax.experimental import pallas as pl
from jax.experimental.pallas import tpu as pltpu
from jax.experimental.pallas import tpu_sc as plsc

_N, _L, _C = 128, 8192, 20
_ROWS = _N * _L
_NW = 32
_CHUNK = 2048

_SC_NCHUNK = 4
_SC_ROWS = _NW * _SC_NCHUNK * _CHUNK
_ROWS_PER_W = _SC_ROWS // _NW
_GROUPS = _CHUNK // 16

_BLKR = 64
_TC_ROWS = _ROWS - _SC_ROWS
_TC_CHUNKS = _TC_ROWS // (_BLKR * 128)

_V0, _V1, _V2 = 0.0024, 0.0763, 0.8426
_IZ_EDGE0 = float(1.0 / np.float32(0.9213))
_IZ_EDGE1 = float(1.0 / np.float32(0.9976))


def _smooth_table() -> np.ndarray:
    vals = np.array([_V0, _V1, _V2, _V1, _V0], dtype=np.float32)
    tab = np.zeros((20, 21), dtype=np.float32)
    for i in range(20):
        for k in range(5):
            p = i + k - 2
            if 0 <= p < 20:
                tab[i, p] += vals[k]
    return tab / tab.sum(axis=1, keepdims=True)


_TABLE = np.zeros(8192, dtype=np.float32)
_TABLE[:6720] = np.tile(_smooth_table().reshape(-1, 1), (1, 16)).reshape(-1)


@functools.partial(
    pl.kernel,
    out_type=jax.ShapeDtypeStruct((_NW * 16,), jnp.float32),
    mesh=plsc.VectorSubcoreMesh(core_axis_name="c", subcore_axis_name="s"),
    compiler_params=pltpu.CompilerParams(needs_layout_passes=False),
    scratch_types=[
        pltpu.VMEM((_C * _CHUNK // 128, 128), jnp.float32),
        pltpu.VMEM((_C * _CHUNK // 128, 128), jnp.float32),
        pltpu.VMEM((_CHUNK // 128, 128), jnp.float32),
        pltpu.VMEM((_CHUNK // 128, 128), jnp.float32),
        pltpu.VMEM((8192,), jnp.float32),
        pltpu.VMEM((16,), jnp.float32),
        pltpu.SemaphoreType.DMA,
        pltpu.SemaphoreType.DMA,
        pltpu.SemaphoreType.DMA,
        pltpu.SemaphoreType.DMA,
    ],
)
def _edm_sc(x_hbm, t_hbm, tab_hbm, out_hbm,
            xba, xbb, tba, tbb, tab_v, acc_v,
            sxa, sxb, sta, stb):
    wid = lax.axis_index("s") * 2 + lax.axis_index("c")
    base_row = wid * _ROWS_PER_W
    lane = lax.broadcasted_iota(jnp.int32, (16,), 0)

    def start(ch, xbuf, tbuf, sx, st):
        row0 = base_row + ch * _CHUNK
        for c in range(_C):
            off = pl.multiple_of((c * _ROWS + row0) // 128, 16)
            pltpu.make_async_copy(
                x_hbm.at[pl.ds(off, _CHUNK // 128), :],
                xbuf.at[pl.ds(c * (_CHUNK // 128), _CHUNK // 128), :],
                sx).start()
        pltpu.make_async_copy(
            t_hbm.at[pl.ds(pl.multiple_of(row0 // 128, 16), _CHUNK // 128), :],
            tbuf, st).start()

    def wait(xbuf, tbuf, sx, st):
        pltpu.make_async_copy(
            x_hbm.at[pl.ds(0, _C * _CHUNK // 128), :], xbuf, sx).wait()
        pltpu.make_async_copy(
            t_hbm.at[pl.ds(0, _CHUNK // 128), :], tbuf, st).wait()

    def compute(xbuf, tbuf):
        @pl.loop(0, _GROUPS)
        def _(g):
            grow = g // 8
            gcol = (g % 8) * 16
            t = tbuf[grow, pl.ds(gcol, 16)]
            f = t * 100.0 - 100.0
            xi = f.astype(jnp.int32)
            xi = jnp.minimum(jnp.maximum(xi, 0), 399)
            bin336 = jnp.right_shift(xi * 3277, 16) * (21 * 16) + lane
            run = jnp.zeros((16,), jnp.float32)
            ssq = jnp.zeros((16,), jnp.float32)
            for c in range(_C):
                xc = xbuf[c * (_CHUNK // 128) + grow, pl.ds(gcol, 16)]
                sc = plsc.load_gather(tab_v, [bin336 + 16 * c])
                run = run + (xc - sc)
                ssq = ssq + run * run
            y = ssq * (1.0 / 20.0) + 1e-6
            r = plsc.bitcast(
                jnp.int32(0x5F3759DF)
                - jnp.right_shift(plsc.bitcast(y, jnp.int32), 1),
                jnp.float32)
            for _ in range(3):
                r = r * (1.5 - 0.5 * y * r * r)
            acc_v[...] = acc_v[...] + y * r

    pltpu.sync_copy(tab_hbm, tab_v)
    acc_v[...] = jnp.zeros((16,), jnp.float32)
    start(0, xba, tba, sxa, sta)

    @pl.loop(0, _SC_NCHUNK, step=2)
    def _(ch):
        start(ch + 1, xbb, tbb, sxb, stb)
        wait(xba, tba, sxa, sta)
        compute(xba, tba)

        @pl.when(ch + 2 < _SC_NCHUNK)
        def _():
            start(ch + 2, xba, tba, sxa, sta)

        wait(xbb, tbb, sxb, stb)
        compute(xbb, tbb)

    pltpu.sync_copy(acc_v, out_hbm.at[pl.ds(wid * 16, 16)])


def _tc_body(*refs):
    x_refs = refs[:_C]
    t_ref = refs[_C]
    out_ref = refs[_C + 1]
    i = pl.program_id(0)

    t = t_ref[...]
    f = t * 100.0 - 100.0
    xi = f.astype(jnp.int32)
    xi = jnp.minimum(jnp.maximum(xi, 0), 399)
    b = jnp.right_shift(xi * 3277, 16)
    inz = jnp.where(jnp.logical_or(b == 0, b == 19), _IZ_EDGE0, 1.0)
    inz = jnp.where(jnp.logical_or(b == 1, b == 18), _IZ_EDGE1, inz)

    run = jnp.zeros((_BLKR, 128), jnp.float32)
    ssq = jnp.zeros((_BLKR, 128), jnp.float32)
    for c in range(_C):
        a = jnp.abs(c - b)
        s = jnp.where(a == 0, _V2, 0.0)
        s = jnp.where(a == 1, _V1, s)
        s = jnp.where(a == 2, _V0, s)
        run = run + (x_refs[c][...] - s * inz)
        ssq = ssq + run * run

    y = ssq * (1.0 / 20.0) + 1e-6
    val = jnp.sqrt(y)

    @pl.when(i == 0)
    def _():
        out_ref[...] = jnp.zeros_like(out_ref)

    out_ref[...] = out_ref[...] + val


_edm_tc = pl.pallas_call(
    _tc_body,
    grid=(_TC_CHUNKS,),
    in_specs=[
        pl.BlockSpec((_BLKR, 128),
                     functools.partial(
                         lambda c_, i: (c_ * (_ROWS // (128 * _BLKR))
                                        + _SC_ROWS // (128 * _BLKR) + i, 0),
                         c))
        for c in range(_C)
    ] + [
        pl.BlockSpec((_BLKR, 128),
                     lambda i: (_SC_ROWS // (128 * _BLKR) + i, 0)),
    ],
    out_specs=pl.BlockSpec((_BLKR, 128), lambda i: (0, 0)),
    out_shape=jax.ShapeDtypeStruct((_BLKR, 128), jnp.float32),
)


def kernel(input, target):
    x = input.transpose(2, 0, 1).reshape(_C * _ROWS // 128, 128)
    t = target.reshape(_ROWS // 128, 128)
    parts_sc = _edm_sc(x, t, jnp.asarray(_TABLE))
    parts_tc = _edm_tc(*([x] * _C), t)
    return (jnp.sum(parts_sc) + jnp.sum(parts_tc)) * (1.0 / _ROWS)


# speedup vs baseline: 1.0660x; 1.0660x over previous
import functools

import numpy as np
import jax
import jax.numpy as jnp
from jax import lax
from jax.experimental import pallas as pl
from jax.experimental.pallas import tpu as pltpu
from jax.experimental.pallas import tpu_sc as plsc

_N, _L, _C = 128, 8192, 20
_ROWS = _N * _L
_NW = 32
_CHUNK = 2048

_SC_NCHUNK = 6
_SC_ROWS = _NW * _SC_NCHUNK * _CHUNK
_ROWS_PER_W = _SC_ROWS // _NW
_GROUPS = _CHUNK // 16

_BLKR = 64
_TC_ROWS = _ROWS - _SC_ROWS
_TC_CHUNKS = _TC_ROWS // (_BLKR * 128)

_V0, _V1, _V2 = 0.0024, 0.0763, 0.8426
_IZ_EDGE0 = float(1.0 / np.float32(0.9213))
_IZ_EDGE1 = float(1.0 / np.float32(0.9976))


def _smooth_table() -> np.ndarray:
    vals = np.array([_V0, _V1, _V2, _V1, _V0], dtype=np.float32)
    tab = np.zeros((20, 21), dtype=np.float32)
    for i in range(20):
        for k in range(5):
            p = i + k - 2
            if 0 <= p < 20:
                tab[i, p] += vals[k]
    return tab / tab.sum(axis=1, keepdims=True)


_TABLE = np.zeros(8192, dtype=np.float32)
_TABLE[:6720] = np.tile(_smooth_table().reshape(-1, 1), (1, 16)).reshape(-1)


@functools.partial(
    pl.kernel,
    out_type=jax.ShapeDtypeStruct((_NW * 16,), jnp.float32),
    mesh=plsc.VectorSubcoreMesh(core_axis_name="c", subcore_axis_name="s"),
    compiler_params=pltpu.CompilerParams(needs_layout_passes=False),
    scratch_types=[
        pltpu.VMEM((_C * _CHUNK // 128, 128), jnp.float32),
        pltpu.VMEM((_C * _CHUNK // 128, 128), jnp.float32),
        pltpu.VMEM((_CHUNK // 128, 128), jnp.float32),
        pltpu.VMEM((_CHUNK // 128, 128), jnp.float32),
        pltpu.VMEM((8192,), jnp.float32),
        pltpu.VMEM((16,), jnp.float32),
        pltpu.SemaphoreType.DMA,
        pltpu.SemaphoreType.DMA,
        pltpu.SemaphoreType.DMA,
        pltpu.SemaphoreType.DMA,
    ],
)
def _edm_sc(x_hbm, t_hbm, tab_hbm, out_hbm,
            xba, xbb, tba, tbb, tab_v, acc_v,
            sxa, sxb, sta, stb):
    wid = lax.axis_index("s") * 2 + lax.axis_index("c")
    base_row = wid * _ROWS_PER_W
    lane = lax.broadcasted_iota(jnp.int32, (16,), 0)

    def start(ch, xbuf, tbuf, sx, st):
        row0 = base_row + ch * _CHUNK
        for c in range(_C):
            off = pl.multiple_of((c * _ROWS + row0) // 128, 16)
            pltpu.make_async_copy(
                x_hbm.at[pl.ds(off, _CHUNK // 128), :],
                xbuf.at[pl.ds(c * (_CHUNK // 128), _CHUNK // 128), :],
                sx).start()
        pltpu.make_async_copy(
            t_hbm.at[pl.ds(pl.multiple_of(row0 // 128, 16), _CHUNK // 128), :],
            tbuf, st).start()

    def wait(xbuf, tbuf, sx, st):
        pltpu.make_async_copy(
            x_hbm.at[pl.ds(0, _C * _CHUNK // 128), :], xbuf, sx).wait()
        pltpu.make_async_copy(
            t_hbm.at[pl.ds(0, _CHUNK // 128), :], tbuf, st).wait()

    def compute(xbuf, tbuf):
        @pl.loop(0, _GROUPS)
        def _(g):
            grow = g // 8
            gcol = (g % 8) * 16
            t = tbuf[grow, pl.ds(gcol, 16)]
            f = t * 100.0 - 100.0
            xi = f.astype(jnp.int32)
            xi = jnp.minimum(jnp.maximum(xi, 0), 399)
            bin336 = jnp.right_shift(xi * 3277, 16) * (21 * 16) + lane
            run = jnp.zeros((16,), jnp.float32)
            ssq = jnp.zeros((16,), jnp.float32)
            for c in range(_C):
                xc = xbuf[c * (_CHUNK // 128) + grow, pl.ds(gcol, 16)]
                sc = plsc.load_gather(tab_v, [bin336 + 16 * c])
                run = run + (xc - sc)
                ssq = ssq + run * run
            y = ssq * (1.0 / 20.0) + 1e-6
            r = plsc.bitcast(
                jnp.int32(0x5F3759DF)
                - jnp.right_shift(plsc.bitcast(y, jnp.int32), 1),
                jnp.float32)
            for _ in range(3):
                r = r * (1.5 - 0.5 * y * r * r)
            acc_v[...] = acc_v[...] + y * r

    pltpu.sync_copy(tab_hbm, tab_v)
    acc_v[...] = jnp.zeros((16,), jnp.float32)
    start(0, xba, tba, sxa, sta)

    @pl.loop(0, _SC_NCHUNK, step=2)
    def _(ch):
        start(ch + 1, xbb, tbb, sxb, stb)
        wait(xba, tba, sxa, sta)
        compute(xba, tba)

        @pl.when(ch + 2 < _SC_NCHUNK)
        def _():
            start(ch + 2, xba, tba, sxa, sta)

        wait(xbb, tbb, sxb, stb)
        compute(xbb, tbb)

    pltpu.sync_copy(acc_v, out_hbm.at[pl.ds(wid * 16, 16)])


def _tc_body(*refs):
    x_refs = refs[:_C]
    t_ref = refs[_C]
    out_ref = refs[_C + 1]
    i = pl.program_id(0)

    t = t_ref[...]
    f = t * 100.0 - 100.0
    xi = f.astype(jnp.int32)
    xi = jnp.minimum(jnp.maximum(xi, 0), 399)
    b = jnp.right_shift(xi * 3277, 16)
    inz = jnp.where(jnp.logical_or(b == 0, b == 19), _IZ_EDGE0, 1.0)
    inz = jnp.where(jnp.logical_or(b == 1, b == 18), _IZ_EDGE1, inz)

    run = jnp.zeros((_BLKR, 128), jnp.float32)
    ssq = jnp.zeros((_BLKR, 128), jnp.float32)
    for c in range(_C):
        a = jnp.abs(c - b)
        s = jnp.where(a == 0, _V2, 0.0)
        s = jnp.where(a == 1, _V1, s)
        s = jnp.where(a == 2, _V0, s)
        run = run + (x_refs[c][...] - s * inz)
        ssq = ssq + run * run

    y = ssq * (1.0 / 20.0) + 1e-6
    val = jnp.sqrt(y)

    @pl.when(i == 0)
    def _():
        out_ref[...] = jnp.zeros_like(out_ref)

    out_ref[...] = out_ref[...] + val


_edm_tc = pl.pallas_call(
    _tc_body,
    grid=(_TC_CHUNKS,),
    in_specs=[
        pl.BlockSpec((_BLKR, 128),
                     functools.partial(
                         lambda c_, i: (c_ * (_ROWS // (128 * _BLKR))
                                        + _SC_ROWS // (128 * _BLKR) + i, 0),
                         c))
        for c in range(_C)
    ] + [
        pl.BlockSpec((_BLKR, 128),
                     lambda i: (_SC_ROWS // (128 * _BLKR) + i, 0)),
    ],
    out_specs=pl.BlockSpec((_BLKR, 128), lambda i: (0, 0)),
    out_shape=jax.ShapeDtypeStruct((_BLKR, 128), jnp.float32),
)


def kernel(input, target):
    x = input.transpose(2, 0, 1).reshape(_C * _ROWS // 128, 128)
    t = target.reshape(_ROWS // 128, 128)
    parts_sc = _edm_sc(x, t, jnp.asarray(_TABLE))
    parts_tc = _edm_tc(*([x] * _C), t)
    return (jnp.sum(parts_sc) + jnp.sum(parts_tc)) * (1.0 / _ROWS)
